# baseline (speedup 1.0000x reference)
import jax
import jax.numpy as jnp
from jax.experimental import pallas as pl

_BINS_SCALE = 10 - 0.0001
_N = 4194304
_ROWS = 4096
_COLS = 1024
_BLOCK_ROWS = 256


def _body(x_ref, t_ref, o_ref):
    g = jnp.abs(jax.nn.sigmoid(x_ref[...]) - t_ref[...])
    o_ref[...] = (g * _BINS_SCALE).astype(jnp.int32)


def kernel(x, target):
    x2 = x.reshape(_ROWS, _COLS)
    t2 = target.reshape(_ROWS, _COLS)
    out = pl.pallas_call(
        _body,
        grid=(_ROWS // _BLOCK_ROWS,),
        in_specs=[
            pl.BlockSpec((_BLOCK_ROWS, _COLS), lambda i: (i, 0)),
            pl.BlockSpec((_BLOCK_ROWS, _COLS), lambda i: (i, 0)),
        ],
        out_specs=pl.BlockSpec((_BLOCK_ROWS, _COLS), lambda i: (i, 0)),
        out_shape=jax.ShapeDtypeStruct((_ROWS, _COLS), jnp.int32),
    )(x2, t2)
    return out.reshape(_N)


# speedup vs baseline: 4.0330x; 4.0330x over previous
import jax
import jax.numpy as jnp
from jax.experimental import pallas as pl

_BINS_SCALE = 10 - 0.0001
_N = 4194304
_ROWS = 4096
_COLS = 1024
_BLOCK_ROWS = 256


_BLOCK = 262144
_GRID = _N // _BLOCK


def _body(x_ref, t_ref, o_ref):
    g = jnp.abs(jax.nn.sigmoid(x_ref[...]) - t_ref[...])
    o_ref[...] = (g * _BINS_SCALE).astype(jnp.int32)


def kernel(x, target):
    return pl.pallas_call(
        _body,
        grid=(_GRID,),
        in_specs=[
            pl.BlockSpec((_BLOCK,), lambda i: (i,)),
            pl.BlockSpec((_BLOCK,), lambda i: (i,)),
        ],
        out_specs=pl.BlockSpec((_BLOCK,), lambda i: (i,)),
        out_shape=jax.ShapeDtypeStruct((_N,), jnp.int32),
    )(x, target)
